# baseline (speedup 1.0000x reference)
import functools

import jax
import jax.numpy as jnp
from jax.experimental import pallas as pl
from jax.experimental.pallas import tpu as pltpu

_P = 256
_D = 768
_V = 49408
_VB = 3088


def _half_update(a, a2, c, base, minval_ref, minidx_ref):
    b2 = jnp.sum(c * c, axis=1, keepdims=True)
    s = jax.lax.dot_general(
        a, c, (((1,), (1,)), ((), ())),
        preferred_element_type=jnp.float32,
        precision=jax.lax.Precision.DEFAULT,
    )
    d2 = (a2 + b2.T) - 2.0 * s

    bmin = jnp.min(d2, axis=1, keepdims=True)
    bidx = jnp.argmin(d2, axis=1).astype(jnp.int32)
    bidx = bidx.reshape(_P, 1) + base

    upd = bmin < minval_ref[...]
    minidx_ref[...] = jnp.where(upd, bidx, minidx_ref[...])
    minval_ref[...] = jnp.where(upd, bmin, minval_ref[...])


def _argmin_kernel(a_ref, c0_ref, c1_ref, ids_ref, minval_ref, minidx_ref,
                   *, n_blocks):
    j = pl.program_id(0)

    @pl.when(j == 0)
    def _init():
        minval_ref[...] = jnp.full((_P, 1), jnp.inf, dtype=jnp.float32)
        minidx_ref[...] = jnp.zeros((_P, 1), dtype=jnp.int32)

    a = a_ref[...]
    a2 = jnp.sum(a * a, axis=1, keepdims=True)
    _half_update(a, a2, c0_ref[...], (2 * j) * _VB, minval_ref, minidx_ref)
    _half_update(a, a2, c1_ref[...], (2 * j + 1) * _VB, minval_ref, minidx_ref)

    @pl.when(j == n_blocks - 1)
    def _done():
        ids_ref[...] = minidx_ref[...]


def _nearest_ids(prompt_embs, clip_embs):
    n_blocks = _V // (2 * _VB)
    ids = pl.pallas_call(
        functools.partial(_argmin_kernel, n_blocks=n_blocks),
        grid=(n_blocks,),
        in_specs=[
            pl.BlockSpec((_P, _D), lambda j: (0, 0)),
            pl.BlockSpec((_VB, _D), lambda j: (2 * j, 0)),
            pl.BlockSpec((_VB, _D), lambda j: (2 * j + 1, 0)),
        ],
        out_specs=pl.BlockSpec((_P, 1), lambda j: (0, 0)),
        out_shape=jax.ShapeDtypeStruct((_P, 1), jnp.int32),
        scratch_shapes=[
            pltpu.VMEM((_P, 1), jnp.float32),
            pltpu.VMEM((_P, 1), jnp.int32),
        ],
        compiler_params=pltpu.CompilerParams(
            dimension_semantics=("arbitrary",),
        ),
    )(prompt_embs, clip_embs, clip_embs)
    return ids.reshape(_P)


@jax.jit
def kernel(prompt_embs, clip_embs):
    ids = _nearest_ids(prompt_embs, clip_embs)
    return (prompt_embs, prompt_embs, ids)


# speedup vs baseline: 1.1116x; 1.1116x over previous
import functools

import jax
import jax.numpy as jnp
from jax.experimental import pallas as pl
from jax.experimental.pallas import tpu as pltpu

_P = 256
_D = 768
_V = 49408
_VB = 6176


def _nn_kernel(a_ref, c_ref, idx_ref, acc_val, acc_idx, *, n_blocks):
    j = pl.program_id(0)

    @pl.when(j == 0)
    def _init():
        acc_val[...] = jnp.full((1, _P), jnp.inf, dtype=jnp.float32)
        acc_idx[...] = jnp.zeros((1, _P), dtype=jnp.int32)

    c = c_ref[...]
    a = a_ref[...]
    dot = jax.lax.dot_general(
        c, a, (((1,), (1,)), ((), ())),
        preferred_element_type=jnp.float32)
    b2 = jnp.sum(c * c, axis=1, keepdims=True)
    s = b2 - 2.0 * dot

    m = jnp.min(s, axis=0, keepdims=True)
    iota = jax.lax.broadcasted_iota(jnp.int32, (_VB, _P), 0)
    li = jnp.min(jnp.where(s == m, iota, _V), axis=0, keepdims=True)
    gi = li + j * _VB

    better = m < acc_val[...]
    acc_val[...] = jnp.where(better, m, acc_val[...])
    acc_idx[...] = jnp.where(better, gi, acc_idx[...])

    @pl.when(j == n_blocks - 1)
    def _done():
        idx_ref[...] = acc_idx[...]


def _nearest_ids(prompt_embs, clip_embs):
    n_blocks = _V // _VB
    ids2d = pl.pallas_call(
        functools.partial(_nn_kernel, n_blocks=n_blocks),
        grid=(n_blocks,),
        in_specs=[
            pl.BlockSpec((_P, _D), lambda j: (0, 0)),
            pl.BlockSpec((_VB, _D), lambda j: (j, 0)),
        ],
        out_specs=pl.BlockSpec((1, _P), lambda j: (0, 0)),
        out_shape=jax.ShapeDtypeStruct((1, _P), jnp.int32),
        scratch_shapes=[
            pltpu.VMEM((1, _P), jnp.float32),
            pltpu.VMEM((1, _P), jnp.int32),
        ],
        compiler_params=pltpu.CompilerParams(
            dimension_semantics=("arbitrary",),
        ),
    )(prompt_embs, clip_embs)
    return ids2d.reshape(_P)


@jax.jit
def kernel(prompt_embs, clip_embs):
    ids = _nearest_ids(prompt_embs, clip_embs)
    return (prompt_embs, prompt_embs, ids)
